# baseline (speedup 1.0000x reference)
import jax
import jax.numpy as jnp
from jax.experimental import pallas as pl
from jax.experimental.pallas import tpu as pltpu


def _make_kernel(num, h, w, size, n_chunks, ch_rows):
    gt_rows = ch_rows // size

    def body(w_ref, g_ref, m0_ref, m1_ref, out_ref, gt_ref):
        j = pl.program_id(1)

        g = g_ref[...]
        r = g.reshape(gt_rows, size, w * size).sum(axis=1)
        p = r.reshape(gt_rows, w, size).sum(axis=2)
        gt_ref[pl.ds(j * gt_rows, gt_rows), :] = p

        @pl.when(j == n_chunks - 1)
        def _():
            gt = gt_ref[...]
            m0 = m0_ref[...]
            m1 = m1_ref[...]
            err0 = jnp.abs(gt - m0)
            err1 = jnp.abs(gt - m1)
            bits0 = jax.lax.bitcast_convert_type(err0, jnp.int32)
            bits1 = jax.lax.bitcast_convert_type(err1, jnp.int32)

            def kth_largest_bits(bits):
                lo = jnp.zeros((), jnp.int32)
                hi = jnp.full((), 0x7F800000, jnp.int32)

                def step(_, carry):
                    lo, hi = carry
                    mid = lo + ((hi - lo) >> 1)
                    cnt = jnp.sum((bits >= mid).astype(jnp.int32))
                    ge = cnt >= num
                    return (jnp.where(ge, mid, lo), jnp.where(ge, hi, mid))

                lo, hi = jax.lax.fori_loop(0, 31, step, (lo, hi))
                return lo

            vmin0 = jax.lax.bitcast_convert_type(kth_largest_bits(bits0),
                                                 jnp.float32)
            vmin1 = jax.lax.bitcast_convert_type(kth_largest_bits(bits1),
                                                 jnp.float32)

            wgt = w_ref[0, 0]
            comb0 = wgt * m0 + (1.0 - wgt) * gt
            comb1 = wgt * m1 + (1.0 - wgt) * gt
            sup0 = jnp.where(err0 >= vmin0, comb1, gt)
            sup1 = jnp.where(err1 >= vmin1, comb0, gt)
            part = jnp.sum((m0 - sup0) ** 2) + jnp.sum((m1 - sup1) ** 2)

            @pl.when(pl.program_id(0) == 0)
            def _():
                out_ref[0, 0] = 0.0

            out_ref[0, 0] += part

    return body


def kernel(dmap_conv, dmap_tran, gt_density, process):
    b, c, h, w = dmap_conv.shape
    gb, gc, gh, gw = gt_density.shape
    size = gh // h
    max_noisy_ratio = 0.1
    max_weight_ratio = 1.0
    num = int(h * w * max_noisy_ratio * 1.0)
    weight = (jnp.asarray(process, jnp.float32) * max_weight_ratio
              ).reshape(1, 1)

    m0 = dmap_conv.reshape(b * h, w)
    m1 = dmap_tran.reshape(b * h, w)
    g2 = gt_density.reshape(gb * gh, gw)

    ch_rows = 64
    n_chunks = gh // ch_rows
    grid = (b, n_chunks)

    out = pl.pallas_call(
        _make_kernel(num, h, w, size, n_chunks, ch_rows),
        grid=grid,
        in_specs=[
            pl.BlockSpec(memory_space=pltpu.SMEM),
            pl.BlockSpec((ch_rows, gw), lambda i, j: (i * n_chunks + j, 0)),
            pl.BlockSpec((h, w), lambda i, j: (i, 0)),
            pl.BlockSpec((h, w), lambda i, j: (i, 0)),
        ],
        out_specs=pl.BlockSpec(memory_space=pltpu.SMEM),
        out_shape=jax.ShapeDtypeStruct((1, 1), jnp.float32),
        scratch_shapes=[pltpu.VMEM((h, w), jnp.float32)],
    )(weight, g2, m0, m1)
    return out.reshape(())


# speedup vs baseline: 1.2087x; 1.2087x over previous
import jax
import jax.numpy as jnp
from jax.experimental import pallas as pl
from jax.experimental.pallas import tpu as pltpu


def _make_kernel(num, h, w, size):
    def body(w_ref, g_ref, m0_ref, m1_ref, out_ref):
        s2 = (jax.lax.broadcasted_iota(jnp.int32, (w * size, w), 0) // size
              == jax.lax.broadcasted_iota(jnp.int32, (w * size, w), 1)
              ).astype(jnp.float32)
        cp = jnp.dot(g_ref[...], s2, preferred_element_type=jnp.float32)
        gt = cp.reshape(h, size, w).sum(axis=1)

        m0 = m0_ref[...]
        m1 = m1_ref[...]
        err0 = jnp.abs(gt - m0)
        err1 = jnp.abs(gt - m1)
        bits0 = jax.lax.bitcast_convert_type(err0, jnp.int32)
        bits1 = jax.lax.bitcast_convert_type(err1, jnp.int32)

        def kth_largest_bits(bits):
            lo = jnp.zeros((), jnp.int32)
            hi = jnp.full((), 0x7F800000, jnp.int32)

            def step(_, carry):
                lo, hi = carry
                mid = lo + ((hi - lo) >> 1)
                cnt = jnp.sum((bits >= mid).astype(jnp.int32))
                ge = cnt >= num
                return (jnp.where(ge, mid, lo), jnp.where(ge, hi, mid))

            lo, hi = jax.lax.fori_loop(0, 31, step, (lo, hi))
            return lo

        vmin0 = jax.lax.bitcast_convert_type(kth_largest_bits(bits0),
                                             jnp.float32)
        vmin1 = jax.lax.bitcast_convert_type(kth_largest_bits(bits1),
                                             jnp.float32)

        wgt = w_ref[0, 0]
        comb0 = wgt * m0 + (1.0 - wgt) * gt
        comb1 = wgt * m1 + (1.0 - wgt) * gt
        sup0 = jnp.where(err0 >= vmin0, comb1, gt)
        sup1 = jnp.where(err1 >= vmin1, comb0, gt)
        part = jnp.sum((m0 - sup0) ** 2) + jnp.sum((m1 - sup1) ** 2)

        @pl.when(pl.program_id(0) == 0)
        def _():
            out_ref[0, 0] = 0.0

        out_ref[0, 0] += part

    return body


def kernel(dmap_conv, dmap_tran, gt_density, process):
    b, c, h, w = dmap_conv.shape
    gb, gc, gh, gw = gt_density.shape
    size = gh // h
    max_noisy_ratio = 0.1
    max_weight_ratio = 1.0
    num = int(h * w * max_noisy_ratio * 1.0)
    weight = (jnp.asarray(process, jnp.float32) * max_weight_ratio
              ).reshape(1, 1)

    m0 = dmap_conv.reshape(b * h, w)
    m1 = dmap_tran.reshape(b * h, w)
    g2 = gt_density.reshape(gb * gh, gw)

    out = pl.pallas_call(
        _make_kernel(num, h, w, size),
        grid=(b,),
        in_specs=[
            pl.BlockSpec(memory_space=pltpu.SMEM),
            pl.BlockSpec((gh, gw), lambda i: (i, 0)),
            pl.BlockSpec((h, w), lambda i: (i, 0)),
            pl.BlockSpec((h, w), lambda i: (i, 0)),
        ],
        out_specs=pl.BlockSpec(memory_space=pltpu.SMEM),
        out_shape=jax.ShapeDtypeStruct((1, 1), jnp.float32),
    )(weight, g2, m0, m1)
    return out.reshape(())


# speedup vs baseline: 10.4620x; 8.6554x over previous
import jax
import jax.numpy as jnp
from jax.experimental import pallas as pl
from jax.experimental.pallas import tpu as pltpu


def _make_kernel(num, b, h, w, size):
    def body(w_ref, g_ref, m0_ref, m1_ref, out_ref, gt_ref):
        i = pl.program_id(0)

        @pl.when(i < b)
        def _pool():
            s2 = (jax.lax.broadcasted_iota(jnp.int32, (w * size, w), 0)
                  // size
                  == jax.lax.broadcasted_iota(jnp.int32, (w * size, w), 1)
                  ).astype(jnp.float32)
            cp = jnp.dot(g_ref[...], s2, preferred_element_type=jnp.float32)
            gt_ref[pl.ds(i * h, h), :] = cp.reshape(h, size, w).sum(axis=1)

        @pl.when(i == b)
        def _loss():
            gt = gt_ref[...]
            m0 = m0_ref[...]
            m1 = m1_ref[...]
            err = jnp.concatenate([jnp.abs(gt - m0), jnp.abs(gt - m1)],
                                  axis=0).reshape(2 * b, h, w)

            lo = jnp.zeros((2 * b, 1, 1), jnp.int32)
            hi = jnp.full((2 * b, 1, 1), 0x7F800000, jnp.int32)

            def step(_, carry):
                lo, hi = carry
                mid = lo + ((hi - lo) >> 1)
                midf = jax.lax.bitcast_convert_type(mid, jnp.float32)
                cnt = jnp.sum((err >= midf).astype(jnp.int32), axis=(1, 2),
                              keepdims=True)
                ge = cnt >= num
                return (jnp.where(ge, mid, lo), jnp.where(ge, hi, mid))

            lo, hi = jax.lax.fori_loop(0, 31, step, (lo, hi))
            vmin = jax.lax.bitcast_convert_type(lo, jnp.float32)

            wgt = w_ref[0, 0]
            gt3 = gt.reshape(b, h, w)
            m03 = m0.reshape(b, h, w)
            m13 = m1.reshape(b, h, w)
            comb0 = wgt * m03 + (1.0 - wgt) * gt3
            comb1 = wgt * m13 + (1.0 - wgt) * gt3
            gt_s = jnp.concatenate([gt3, gt3], axis=0)
            m_s = jnp.concatenate([m03, m13], axis=0)
            comb_other = jnp.concatenate([comb1, comb0], axis=0)
            sup = jnp.where(err >= vmin, comb_other, gt_s)
            out_ref[0, 0] = jnp.sum((m_s - sup) ** 2)

    return body


def kernel(dmap_conv, dmap_tran, gt_density, process):
    b, c, h, w = dmap_conv.shape
    gb, gc, gh, gw = gt_density.shape
    size = gh // h
    max_noisy_ratio = 0.1
    max_weight_ratio = 1.0
    num = int(h * w * max_noisy_ratio * 1.0)
    weight = (jnp.asarray(process, jnp.float32) * max_weight_ratio
              ).reshape(1, 1)

    m0 = dmap_conv.reshape(b * h, w)
    m1 = dmap_tran.reshape(b * h, w)
    g2 = gt_density.reshape(gb * gh, gw)

    out = pl.pallas_call(
        _make_kernel(num, b, h, w, size),
        grid=(b + 1,),
        in_specs=[
            pl.BlockSpec(memory_space=pltpu.SMEM),
            pl.BlockSpec((gh, gw), lambda i: (jnp.minimum(i, b - 1), 0)),
            pl.BlockSpec((b * h, w), lambda i: (0, 0)),
            pl.BlockSpec((b * h, w), lambda i: (0, 0)),
        ],
        out_specs=pl.BlockSpec(memory_space=pltpu.SMEM),
        out_shape=jax.ShapeDtypeStruct((1, 1), jnp.float32),
        scratch_shapes=[pltpu.VMEM((b * h, w), jnp.float32)],
    )(weight, g2, m0, m1)
    return out.reshape(())
